# baseline (speedup 1.0000x reference)
import functools

import jax
import jax.numpy as jnp
from jax import lax
from jax.experimental import pallas as pl
from jax.experimental.pallas import tpu as pltpu

_R = 4
_N = 10000
_H = 256
_OUT = 16
_NB = 2

_NPAD = 10240
_BT = 256


def _combine_body(cref, aref, dref, vref, bref, oref, *, relu):
    ab = aref[...]
    inv = 1.0 / jnp.maximum(dref[...], 1.0)
    ho = vref.shape[2]
    out = jnp.zeros((_BT, ho), jnp.float32)
    for b in range(_NB):
        db = jnp.zeros((_BT, _H), jnp.float32)
        for r in range(_R):
            db = db + cref[r, b] * (inv[r][:, None] * ab[r])
        out = out + lax.dot_general(
            db, vref[b], (((1,), (0,)), ((), ())),
            precision=lax.Precision.HIGHEST,
            preferred_element_type=jnp.float32)
    out = out + bref[...]
    if relu:
        out = jnp.maximum(out, 0.0)
    oref[...] = out


def _combine(a, deg, cmat, v, bias, relu):
    ho = v.shape[2]
    body = functools.partial(_combine_body, relu=relu)
    return pl.pallas_call(
        body,
        grid=(_NPAD // _BT,),
        in_specs=[
            pl.BlockSpec(memory_space=pltpu.SMEM),
            pl.BlockSpec((_R, _BT, _H), lambda i: (0, i, 0)),
            pl.BlockSpec((_R, _BT), lambda i: (0, i)),
            pl.BlockSpec((_NB, _H, ho), lambda i: (0, 0, 0)),
            pl.BlockSpec((1, ho), lambda i: (0, 0)),
        ],
        out_specs=pl.BlockSpec((_BT, ho), lambda i: (i, 0)),
        out_shape=jax.ShapeDtypeStruct((_NPAD, ho), jnp.float32),
    )(cmat, a, deg, v, bias)


def kernel(embed, edge_index, V0, C0, b0, V1, C1, b1, V2, C2, b2):
    src = edge_index[:, 0, :]
    dst = edge_index[:, 1, :]

    h = jnp.pad(embed.astype(jnp.float32), ((0, _NPAD - _N), (0, 0)))

    ones = jnp.ones(src.shape[1], jnp.float32)
    deg = jnp.stack([
        jnp.zeros((_NPAD,), jnp.float32).at[dst[r]].add(ones)
        for r in range(_R)
    ])

    v2p = jnp.pad(V2, ((0, 0), (0, 0), (0, 128 - _OUT)))
    b2p = jnp.pad(b2, (0, 128 - _OUT))
    layers = (
        (V0, C0, b0, True),
        (V1, C1, b1, True),
        (v2p, C2, b2p, False),
    )
    for v, cmat, bias, relu in layers:
        a = jnp.stack([
            jnp.zeros((_NPAD, _H), jnp.float32).at[dst[r]].add(h[src[r]])
            for r in range(_R)
        ])
        h = _combine(a, deg, cmat, v, bias.reshape(1, -1), relu)
    return h[:_N, :_OUT]
